# Initial kernel scaffold; baseline (speedup 1.0000x reference)
import jax
import jax.numpy as jnp
from jax.experimental import pallas as pl


def kernel(pl_name_src_tokens, pl_collaborative, pl_duration_bucket, num_pl_songs_bucket, num_pl_artists_bucket, num_pl_albums_bucket, track_uri_pl, track_name_pl_tokens, artist_uri_pl, artist_name_pl_tokens, album_uri_pl, album_name_pl_tokens, artist_genres_pl_tokens, duration_ms_songs_pl_bucket, track_pop_pl_bucket, artist_pop_pl_bucket, artists_followers_pl_bucket, track_danceability_pl_bucket, track_energy_pl_bucket, track_key_pl, track_loudness_pl_bucket, track_mode_pl, emb_pl_name_src, emb_pl_collab, emb_pl_duration, emb_num_songs, emb_num_artists, emb_num_albums, emb_track_uri, emb_track_name, emb_artist_uri, emb_artist_name, emb_album_uri, emb_album_name, emb_artist_genres, emb_dur_songs, emb_track_pop, emb_artist_pop, emb_followers, emb_dance, emb_energy, emb_key, emb_loudness, emb_mode, W1, b1, W2, b2):
    raise NotImplementedError("write your pallas kernel here")


# speedup vs baseline: 5.3256x; 5.3256x over previous
import functools

import jax
import jax.numpy as jnp
from jax import lax
from jax.experimental import pallas as pl
from jax.experimental.pallas import tpu as pltpu
from jax.experimental.pallas import tpu_sc as plsc

B = 4096
D = 64
NC = 2
NS = 16
NW = NC * NS
IPW = B // NW

BIG_KS = (8, 5, 40, 5, 40, 5, 40, 40)
BIG_CHUNK_ITEMS = (64, 128, 16, 128, 16, 128, 16, 16)
BUF_N = {64 * 8: None, 128 * 5: None, 16 * 40: None}
BUF_SIZES = sorted({it * k for it, k in zip(BIG_CHUNK_ITEMS, BIG_KS)})

TB = 512
SMALL_NS = (4, 21, 21, 21, 21)
POOLED_NS = (21, 21, 21, 21, 21, 21, 13, 21, 4)


def _sc_gather_body(*refs):
    tables = refs[0:8]
    idxs = refs[8:16]
    outs = refs[16:24]
    scratch = refs[24:]
    idx_bufs = {n: scratch[i] for i, n in enumerate(BUF_SIZES)}
    row_bufs = {n: scratch[len(BUF_SIZES) + i] for i, n in enumerate(BUF_SIZES)}
    pool_v = scratch[2 * len(BUF_SIZES)]
    sem = scratch[2 * len(BUF_SIZES) + 1]

    c = lax.axis_index("c")
    s = lax.axis_index("s")
    wid = s * NC + c

    for f in range(8):
        k = BIG_KS[f]
        items = BIG_CHUNK_ITEMS[f]
        n = items * k
        idx_v = idx_bufs[n]
        rows_v = row_bufs[n]
        inv = 1.0 / k

        def chunk_body(ch, _, f=f, k=k, items=items, n=n,
                       idx_v=idx_v, rows_v=rows_v, inv=inv):
            it_base = wid * IPW + ch * items
            pltpu.sync_copy(idxs[f].at[pl.ds(it_base * k, n)], idx_v)
            pltpu.async_copy(tables[f].at[idx_v], rows_v, sem).wait()

            def item_body(i, _):
                for q in range(4):
                    acc = rows_v[i * k, pl.ds(q * 16, 16)]
                    for j in range(1, k):
                        acc = acc + rows_v[i * k + j, pl.ds(q * 16, 16)]
                    pool_v[i, pl.ds(q * 16, 16)] = acc * inv
                return 0

            lax.fori_loop(0, items, item_body, 0)
            pltpu.sync_copy(pool_v.at[pl.ds(0, items)],
                            outs[f].at[pl.ds(it_base, items)])
            return 0

        lax.fori_loop(0, IPW // items, chunk_body, 0)


def _sc_gather(tables, idxs):
    mesh = plsc.VectorSubcoreMesh(core_axis_name="c", subcore_axis_name="s",
                                  num_cores=NC, num_subcores=NS)
    scratch = ([pltpu.VMEM((n,), jnp.int32) for n in BUF_SIZES]
               + [pltpu.VMEM((n, D), jnp.float32) for n in BUF_SIZES]
               + [pltpu.VMEM((IPW, D), jnp.float32), pltpu.SemaphoreType.DMA])
    fn = pl.kernel(
        _sc_gather_body,
        out_type=[jax.ShapeDtypeStruct((B, D), jnp.float32) for _ in range(8)],
        mesh=mesh,
        scratch_types=scratch,
        compiler_params=pltpu.CompilerParams(use_tc_tiling_on_sc=False),
    )
    return fn(*tables, *idxs)


def _tc_dense_body(bf0, bf1, bf2, bf3, bf4, bf5, bf6, bf7, sidx,
                   t_collab, t_dur, t_songs, t_artists, t_albums,
                   t_dursongs, t_pop, t_apop, t_fol, t_dance, t_energy,
                   t_key, t_loud, t_mode, w1, b1, w2, b2, out):
    cols = sidx[...]

    def onehot(col, n):
        c = cols[:, col][:, None]
        i = lax.broadcasted_iota(jnp.int32, (TB, n), 1)
        return (c == i).astype(jnp.float32)

    def np_feat(col, table):
        n = table.shape[0]
        return jnp.dot(onehot(col, n), table[...],
                       preferred_element_type=jnp.float32)

    def pooled_feat(col0, table):
        n = table.shape[0]
        h = onehot(col0, n)
        for j in range(1, 5):
            h = h + onehot(col0 + j, n)
        return jnp.dot(h, table[...], preferred_element_type=jnp.float32) * 0.2

    feats = [
        bf0[...],
        np_feat(0, t_collab), np_feat(1, t_dur), np_feat(2, t_songs),
        np_feat(3, t_artists), np_feat(4, t_albums),
        bf1[...], bf2[...], bf3[...], bf4[...], bf5[...], bf6[...], bf7[...],
        pooled_feat(5, t_dursongs), pooled_feat(10, t_pop),
        pooled_feat(15, t_apop), pooled_feat(20, t_fol),
        pooled_feat(25, t_dance), pooled_feat(30, t_energy),
        pooled_feat(35, t_key), pooled_feat(40, t_loud),
        pooled_feat(45, t_mode),
    ]
    x = jnp.concatenate(feats, axis=1)
    h = jnp.dot(x, w1[...], preferred_element_type=jnp.float32) + b1[...]
    h = jnp.maximum(h, 0.0)
    out[...] = jnp.dot(h, w2[...], preferred_element_type=jnp.float32) + b2[...]


def _tc_dense(big_feats, sidx, small_tables, w1, b1, w2, b2):
    grid = (B // TB,)
    bf_spec = pl.BlockSpec((TB, D), lambda i: (i, 0))
    full = lambda arr: pl.BlockSpec(arr.shape, lambda i: (0,) * arr.ndim)
    in_specs = ([bf_spec] * 8
                + [pl.BlockSpec((TB, 50), lambda i: (i, 0))]
                + [full(t) for t in small_tables]
                + [full(w1), full(b1), full(w2), full(b2)])
    return pl.pallas_call(
        _tc_dense_body,
        grid=grid,
        in_specs=in_specs,
        out_specs=pl.BlockSpec((TB, 128), lambda i: (i, 0)),
        out_shape=jax.ShapeDtypeStruct((B, 128), jnp.float32),
    )(*big_feats, sidx, *small_tables, w1, b1, w2, b2)


def kernel(pl_name_src_tokens, pl_collaborative, pl_duration_bucket,
           num_pl_songs_bucket, num_pl_artists_bucket, num_pl_albums_bucket,
           track_uri_pl, track_name_pl_tokens, artist_uri_pl,
           artist_name_pl_tokens, album_uri_pl, album_name_pl_tokens,
           artist_genres_pl_tokens, duration_ms_songs_pl_bucket,
           track_pop_pl_bucket, artist_pop_pl_bucket,
           artists_followers_pl_bucket, track_danceability_pl_bucket,
           track_energy_pl_bucket, track_key_pl, track_loudness_pl_bucket,
           track_mode_pl, emb_pl_name_src, emb_pl_collab, emb_pl_duration,
           emb_num_songs, emb_num_artists, emb_num_albums, emb_track_uri,
           emb_track_name, emb_artist_uri, emb_artist_name, emb_album_uri,
           emb_album_name, emb_artist_genres, emb_dur_songs, emb_track_pop,
           emb_artist_pop, emb_followers, emb_dance, emb_energy, emb_key,
           emb_loudness, emb_mode, W1, b1, W2, b2):
    big_tables = [emb_pl_name_src, emb_track_uri, emb_track_name,
                  emb_artist_uri, emb_artist_name, emb_album_uri,
                  emb_album_name, emb_artist_genres]
    big_idxs = [pl_name_src_tokens.reshape(-1), track_uri_pl.reshape(-1),
                track_name_pl_tokens.reshape(-1), artist_uri_pl.reshape(-1),
                artist_name_pl_tokens.reshape(-1), album_uri_pl.reshape(-1),
                album_name_pl_tokens.reshape(-1),
                artist_genres_pl_tokens.reshape(-1)]
    big_feats = _sc_gather(big_tables, big_idxs)

    sidx = jnp.concatenate(
        [pl_collaborative[:, None], pl_duration_bucket[:, None],
         num_pl_songs_bucket[:, None], num_pl_artists_bucket[:, None],
         num_pl_albums_bucket[:, None], duration_ms_songs_pl_bucket,
         track_pop_pl_bucket, artist_pop_pl_bucket,
         artists_followers_pl_bucket, track_danceability_pl_bucket,
         track_energy_pl_bucket, track_key_pl, track_loudness_pl_bucket,
         track_mode_pl], axis=1)
    small_tables = [emb_pl_collab, emb_pl_duration, emb_num_songs,
                    emb_num_artists, emb_num_albums, emb_dur_songs,
                    emb_track_pop, emb_artist_pop, emb_followers, emb_dance,
                    emb_energy, emb_key, emb_loudness, emb_mode]
    return _tc_dense(big_feats, sidx, small_tables,
                     W1, b1[None, :], W2, b2[None, :])


# speedup vs baseline: 7.3597x; 1.3819x over previous
import functools

import jax
import jax.numpy as jnp
from jax import lax
from jax.experimental import pallas as pl
from jax.experimental.pallas import tpu as pltpu
from jax.experimental.pallas import tpu_sc as plsc

B = 4096
D = 64
NC = 2
NS = 16
NW = NC * NS
IPW = B // NW

BIG_KS = (8, 5, 40, 5, 40, 5, 40, 40)
BIG_CHUNK_ITEMS = (64, 128, 16, 128, 16, 128, 16, 16)

TB = 512
SMALL_NS = (4, 21, 21, 21, 21)
POOLED_NS = (21, 21, 21, 21, 21, 21, 13, 21, 4)


def _make_sc_gather_body(ks, chunk_items, buf_sizes):
    nf = len(ks)

    def body(*refs):
        tables = refs[0:nf]
        idxs = refs[nf:2 * nf]
        outs = refs[2 * nf:3 * nf]
        scratch = refs[3 * nf:]
        idx_bufs = {n: scratch[i] for i, n in enumerate(buf_sizes)}
        row_bufs = {n: scratch[len(buf_sizes) + i] for i, n in enumerate(buf_sizes)}
        pool_v = scratch[2 * len(buf_sizes)]
        sem = scratch[2 * len(buf_sizes) + 1]

        c = lax.axis_index("c")
        s = lax.axis_index("s")
        wid = s * NC + c

        for f in range(nf):
            k = ks[f]
            items = chunk_items[f]
            n = items * k
            idx_v = idx_bufs[n]
            rows_v = row_bufs[n]
            inv = 1.0 / k

            def chunk_body(ch, _, f=f, k=k, items=items, n=n,
                           idx_v=idx_v, rows_v=rows_v, inv=inv):
                it_base = wid * IPW + ch * items
                pltpu.sync_copy(idxs[f].at[pl.ds(it_base * k, n)], idx_v)
                pltpu.async_copy(tables[f].at[idx_v], rows_v, sem).wait()

                def item_body(i, _):
                    for q in range(4):
                        acc = rows_v[i * k, pl.ds(q * 16, 16)]
                        for j in range(1, k):
                            acc = acc + rows_v[i * k + j, pl.ds(q * 16, 16)]
                        pool_v[i, pl.ds(q * 16, 16)] = acc * inv
                    return 0

                lax.fori_loop(0, items, item_body, 0)
                pltpu.sync_copy(pool_v.at[pl.ds(0, items)],
                                outs[f].at[pl.ds(it_base, items)])
                return 0

            lax.fori_loop(0, IPW // items, chunk_body, 0)

    return body


def _sc_gather(tables, idxs, ks, chunk_items):
    buf_sizes = sorted({it * k for it, k in zip(chunk_items, ks)})
    mesh = plsc.VectorSubcoreMesh(core_axis_name="c", subcore_axis_name="s",
                                  num_cores=NC, num_subcores=NS)
    scratch = ([pltpu.VMEM((n,), jnp.int32) for n in buf_sizes]
               + [pltpu.VMEM((n, D), jnp.float32) for n in buf_sizes]
               + [pltpu.VMEM((IPW, D), jnp.float32), pltpu.SemaphoreType.DMA])
    fn = pl.kernel(
        _make_sc_gather_body(ks, chunk_items, buf_sizes),
        out_type=[jax.ShapeDtypeStruct((B, D), jnp.float32)
                  for _ in range(len(ks))],
        mesh=mesh,
        scratch_types=scratch,
        compiler_params=pltpu.CompilerParams(use_tc_tiling_on_sc=False),
    )
    return fn(*tables, *idxs)


def _tc_dense_body(bf0, bf1, bf2, bf3, bf4, bf5, bf6, bf7, sidx,
                   t_collab, t_dur, t_songs, t_artists, t_albums,
                   t_dursongs, t_pop, t_apop, t_fol, t_dance, t_energy,
                   t_key, t_loud, t_mode, w1, b1, w2, b2, out):
    cols = sidx[...]

    def onehot(col, n):
        c = cols[:, col][:, None]
        i = lax.broadcasted_iota(jnp.int32, (TB, n), 1)
        return (c == i).astype(jnp.float32)

    def np_feat(col, table):
        n = table.shape[0]
        return jnp.dot(onehot(col, n), table[...],
                       preferred_element_type=jnp.float32)

    def pooled_feat(col0, table):
        n = table.shape[0]
        h = onehot(col0, n)
        for j in range(1, 5):
            h = h + onehot(col0 + j, n)
        return jnp.dot(h, table[...], preferred_element_type=jnp.float32) * 0.2

    feats = [
        bf0[...],
        np_feat(0, t_collab), np_feat(1, t_dur), np_feat(2, t_songs),
        np_feat(3, t_artists), np_feat(4, t_albums),
        bf1[...], bf2[...], bf3[...], bf4[...], bf5[...], bf6[...], bf7[...],
        pooled_feat(5, t_dursongs), pooled_feat(10, t_pop),
        pooled_feat(15, t_apop), pooled_feat(20, t_fol),
        pooled_feat(25, t_dance), pooled_feat(30, t_energy),
        pooled_feat(35, t_key), pooled_feat(40, t_loud),
        pooled_feat(45, t_mode),
    ]
    x = jnp.concatenate(feats, axis=1)
    h = jnp.dot(x, w1[...], preferred_element_type=jnp.float32) + b1[...]
    h = jnp.maximum(h, 0.0)
    out[...] = jnp.dot(h, w2[...], preferred_element_type=jnp.float32) + b2[...]


def _tc_dense(big_feats, sidx, small_tables, w1, b1, w2, b2):
    grid = (B // TB,)
    bf_spec = pl.BlockSpec((TB, D), lambda i: (i, 0))
    full = lambda arr: pl.BlockSpec(arr.shape, lambda i: (0,) * arr.ndim)
    in_specs = ([bf_spec] * 8
                + [pl.BlockSpec((TB, 50), lambda i: (i, 0))]
                + [full(t) for t in small_tables]
                + [full(w1), full(b1), full(w2), full(b2)])
    return pl.pallas_call(
        _tc_dense_body,
        grid=grid,
        in_specs=in_specs,
        out_specs=pl.BlockSpec((TB, 128), lambda i: (i, 0)),
        out_shape=jax.ShapeDtypeStruct((B, 128), jnp.float32),
    )(*big_feats, sidx, *small_tables, w1, b1, w2, b2)


def kernel(pl_name_src_tokens, pl_collaborative, pl_duration_bucket,
           num_pl_songs_bucket, num_pl_artists_bucket, num_pl_albums_bucket,
           track_uri_pl, track_name_pl_tokens, artist_uri_pl,
           artist_name_pl_tokens, album_uri_pl, album_name_pl_tokens,
           artist_genres_pl_tokens, duration_ms_songs_pl_bucket,
           track_pop_pl_bucket, artist_pop_pl_bucket,
           artists_followers_pl_bucket, track_danceability_pl_bucket,
           track_energy_pl_bucket, track_key_pl, track_loudness_pl_bucket,
           track_mode_pl, emb_pl_name_src, emb_pl_collab, emb_pl_duration,
           emb_num_songs, emb_num_artists, emb_num_albums, emb_track_uri,
           emb_track_name, emb_artist_uri, emb_artist_name, emb_album_uri,
           emb_album_name, emb_artist_genres, emb_dur_songs, emb_track_pop,
           emb_artist_pop, emb_followers, emb_dance, emb_energy, emb_key,
           emb_loudness, emb_mode, W1, b1, W2, b2):
    tok_feats = _sc_gather(
        [emb_pl_name_src, emb_track_name, emb_artist_name, emb_album_name,
         emb_artist_genres],
        [pl_name_src_tokens.reshape(-1), track_name_pl_tokens.reshape(-1),
         artist_name_pl_tokens.reshape(-1), album_name_pl_tokens.reshape(-1),
         artist_genres_pl_tokens.reshape(-1)],
        ks=(8, 40, 40, 40, 40), chunk_items=(64, 16, 16, 16, 16))
    uri_feats = _sc_gather(
        [emb_track_uri, emb_artist_uri, emb_album_uri],
        [track_uri_pl.reshape(-1), artist_uri_pl.reshape(-1),
         album_uri_pl.reshape(-1)],
        ks=(5, 5, 5), chunk_items=(128, 128, 128))
    big_feats = [tok_feats[0], uri_feats[0], tok_feats[1], uri_feats[1],
                 tok_feats[2], uri_feats[2], tok_feats[3], tok_feats[4]]

    sidx = jnp.concatenate(
        [pl_collaborative[:, None], pl_duration_bucket[:, None],
         num_pl_songs_bucket[:, None], num_pl_artists_bucket[:, None],
         num_pl_albums_bucket[:, None], duration_ms_songs_pl_bucket,
         track_pop_pl_bucket, artist_pop_pl_bucket,
         artists_followers_pl_bucket, track_danceability_pl_bucket,
         track_energy_pl_bucket, track_key_pl, track_loudness_pl_bucket,
         track_mode_pl], axis=1)
    small_tables = [emb_pl_collab, emb_pl_duration, emb_num_songs,
                    emb_num_artists, emb_num_albums, emb_dur_songs,
                    emb_track_pop, emb_artist_pop, emb_followers, emb_dance,
                    emb_energy, emb_key, emb_loudness, emb_mode]
    return _tc_dense(big_feats, sidx, small_tables,
                     W1, b1[None, :], W2, b2[None, :])
